# Initial kernel scaffold; baseline (speedup 1.0000x reference)
import jax
import jax.numpy as jnp
from jax.experimental import pallas as pl


def kernel(x, edge_index, W1, b1, W2, b2, W3, b3, g1, be1, g2, be2):
    raise NotImplementedError("write your pallas kernel here")


# speedup vs baseline: 16.7894x; 16.7894x over previous
import functools

import jax
import jax.numpy as jnp
from jax import lax
from jax.experimental import pallas as pl
from jax.experimental.pallas import tpu as pltpu
from jax.experimental.pallas import tpu_sc as plsc

_N = 10000
_E = 320000
_D = 128
_EPS = 1e-5
_NC = 2
_NS = 16
_NW = _NC * _NS
_EPW = _E // _NW
_K = 80
_NCH = _EPW // _K
_RPS = 632
_DEGW = 16


def _row_off(s):
  return pl.multiple_of(jnp.minimum(s * _RPS, _N - _RPS), 8)


def _sc_mesh():
  return plsc.VectorSubcoreMesh(
      core_axis_name="c", subcore_axis_name="s",
      num_cores=_NC, num_subcores=_NS)


def _sc_degree(dst, zeros_nw, ones_kw):
  @functools.partial(
      pl.kernel,
      out_type=jax.ShapeDtypeStruct((_NC, _N, _DEGW), jnp.float32),
      mesh=_sc_mesh(),
      compiler_params=pltpu.CompilerParams(use_tc_tiling_on_sc=False),
      scratch_types=[
          pltpu.VMEM((_NCH, _K), jnp.int32),
          pltpu.VMEM((_K, _DEGW), jnp.float32),
          pltpu.VMEM_SHARED((_N, _DEGW), jnp.float32),
      ],
  )
  def deg_kernel(dst_hbm, zero_hbm, one_hbm, out_hbm, dst_v, one_v, acc_sh):
    c = lax.axis_index("c")
    s = lax.axis_index("s")
    wid = s * _NC + c
    off = _row_off(s)
    pltpu.sync_copy(dst_hbm.at[wid], dst_v)
    pltpu.sync_copy(one_hbm, one_v)
    pltpu.sync_copy(zero_hbm.at[pl.ds(off, _RPS)],
                    acc_sh.at[pl.ds(off, _RPS)])
    plsc.subcore_barrier()

    def body(j, carry):
      pltpu.sync_copy(one_v, acc_sh.at[dst_v.at[j]], add=True)
      return carry

    lax.fori_loop(0, _NCH, body, 0)
    plsc.subcore_barrier()
    pltpu.sync_copy(acc_sh.at[pl.ds(off, _RPS)],
                    out_hbm.at[c, pl.ds(off, _RPS)])

  return deg_kernel(dst, zeros_nw, ones_kw)


def _sc_spmm(hs, src, dst):
  @functools.partial(
      pl.kernel,
      out_type=jax.ShapeDtypeStruct((_NC, _N, _D), jnp.float32),
      mesh=_sc_mesh(),
      scratch_types=[
          pltpu.VMEM((_NCH, _K), jnp.int32),
          pltpu.VMEM((_NCH, _K), jnp.int32),
          pltpu.VMEM((_K, _D), jnp.float32),
          pltpu.VMEM_SHARED((_N, _D), jnp.float32),
          pltpu.SemaphoreType.DMA,
      ],
  )
  def spmm_kernel(hs_hbm, src_hbm, dst_hbm, out_hbm,
                  src_v, dst_v, rows_v, acc_sh, sem):
    c = lax.axis_index("c")
    s = lax.axis_index("s")
    wid = s * _NC + c
    off = _row_off(s)
    pltpu.sync_copy(src_hbm.at[wid], src_v)
    pltpu.sync_copy(dst_hbm.at[wid], dst_v)
    pltpu.sync_copy(hs_hbm.at[pl.ds(off, _RPS)],
                    acc_sh.at[pl.ds(off, _RPS)])
    plsc.subcore_barrier()

    def body(j, carry):
      pltpu.async_copy(hs_hbm.at[src_v.at[j]], rows_v, sem).wait()
      pltpu.sync_copy(rows_v, acc_sh.at[dst_v.at[j]], add=True)
      return carry

    lax.fori_loop(0, _NCH, body, 0)
    plsc.subcore_barrier()
    pltpu.sync_copy(acc_sh.at[pl.ds(off, _RPS)],
                    out_hbm.at[c, pl.ds(off, _RPS)])

  return spmm_kernel(hs, src, dst)


def _tc_pre(x, W1, degs):
  def body(x_ref, w_ref, deg_ref, p_ref, dinv_ref):
    deg = deg_ref[0, :, 0:1] + deg_ref[1, :, 0:1] + 1.0
    dinvb = jnp.broadcast_to(lax.rsqrt(deg), (_N, _D))
    dinv_ref[...] = dinvb
    p_ref[...] = jnp.dot(x_ref[...], w_ref[...],
                         preferred_element_type=jnp.float32) * dinvb

  return pl.pallas_call(
      body,
      out_shape=[jax.ShapeDtypeStruct((_N, _D), jnp.float32),
                 jax.ShapeDtypeStruct((_N, _D), jnp.float32)],
  )(x, W1, degs)


def _tc_mid(accs, P, dinvb, b, g, be, Wn):
  def body(acc_ref, p_ref, dinv_ref, b_ref, g_ref, be_ref, w_ref, out_ref):
    S = acc_ref[0] + acc_ref[1] - p_ref[...]
    T = S * dinv_ref[...] + b_ref[...]
    m = jnp.mean(T, axis=0, keepdims=True)
    v = jnp.mean((T - m) ** 2, axis=0, keepdims=True)
    Hn = g_ref[...] * (T - m) * lax.rsqrt(v + _EPS) + be_ref[...]
    Hn = jnp.maximum(Hn, 0.0)
    out_ref[...] = jnp.dot(Hn, w_ref[...],
                           preferred_element_type=jnp.float32) * dinv_ref[...]

  return pl.pallas_call(
      body,
      out_shape=jax.ShapeDtypeStruct((_N, _D), jnp.float32),
  )(accs, P, dinvb, b, g, be, Wn)


def _tc_fin(accs, P, dinvb, b):
  def body(acc_ref, p_ref, dinv_ref, b_ref, out_ref):
    out_ref[...] = (acc_ref[0] + acc_ref[1] - p_ref[...]) * dinv_ref[...] \
        + b_ref[...]

  return pl.pallas_call(
      body,
      out_shape=jax.ShapeDtypeStruct((_N, _D), jnp.float32),
  )(accs, P, dinvb, b)


def kernel(x, edge_index, W1, b1, W2, b2, W3, b3, g1, be1, g2, be2):
  ei = edge_index.astype(jnp.int32)
  src = ei[0].reshape(_NW, _NCH, _K)
  dst = ei[1].reshape(_NW, _NCH, _K)
  zeros_nw = jnp.zeros((_N, _DEGW), jnp.float32)
  ones_kw = jnp.ones((_K, _DEGW), jnp.float32)

  degs = _sc_degree(dst, zeros_nw, ones_kw)
  P1, dinvb = _tc_pre(x, W1, degs)
  acc1 = _sc_spmm(P1, src, dst)
  P2 = _tc_mid(acc1, P1, dinvb, b1.reshape(1, _D), g1.reshape(1, _D),
               be1.reshape(1, _D), W2)
  acc2 = _sc_spmm(P2, src, dst)
  P3 = _tc_mid(acc2, P2, dinvb, b2.reshape(1, _D), g2.reshape(1, _D),
               be2.reshape(1, _D), W3)
  acc3 = _sc_spmm(P3, src, dst)
  return _tc_fin(acc3, P3, dinvb, b3.reshape(1, _D))


# speedup vs baseline: 21.1661x; 1.2607x over previous
import functools

import jax
import jax.numpy as jnp
from jax import lax
from jax.experimental import pallas as pl
from jax.experimental.pallas import tpu as pltpu
from jax.experimental.pallas import tpu_sc as plsc

_N = 10000
_E = 320000
_D = 128
_EPS = 1e-5
_NC = 2
_NS = 16
_NW = _NC * _NS
_EPW = _E // _NW
_K = 80
_NCH = _EPW // _K
_RPS = 632
_DEGW = 16


def _row_off(s):
  return pl.multiple_of(jnp.minimum(s * _RPS, _N - _RPS), 8)


def _sc_mesh():
  return plsc.VectorSubcoreMesh(
      core_axis_name="c", subcore_axis_name="s",
      num_cores=_NC, num_subcores=_NS)


def _sc_degree(dst, zeros_nw, ones_kw):
  @functools.partial(
      pl.kernel,
      out_type=jax.ShapeDtypeStruct((_NC, _N, _DEGW), jnp.float32),
      mesh=_sc_mesh(),
      compiler_params=pltpu.CompilerParams(use_tc_tiling_on_sc=False),
      scratch_types=[
          pltpu.VMEM((_NCH, _K), jnp.int32),
          pltpu.VMEM((_K, _DEGW), jnp.float32),
          pltpu.VMEM_SHARED((_N, _DEGW), jnp.float32),
      ],
  )
  def deg_kernel(dst_hbm, zero_hbm, one_hbm, out_hbm, dst_v, one_v, acc_sh):
    c = lax.axis_index("c")
    s = lax.axis_index("s")
    wid = s * _NC + c
    off = _row_off(s)
    pltpu.sync_copy(dst_hbm.at[wid], dst_v)
    pltpu.sync_copy(one_hbm, one_v)
    pltpu.sync_copy(zero_hbm.at[pl.ds(off, _RPS)],
                    acc_sh.at[pl.ds(off, _RPS)])
    plsc.subcore_barrier()

    def body(j, carry):
      pltpu.sync_copy(one_v, acc_sh.at[dst_v.at[j]], add=True)
      return carry

    lax.fori_loop(0, _NCH, body, 0)
    plsc.subcore_barrier()
    pltpu.sync_copy(acc_sh.at[pl.ds(off, _RPS)],
                    out_hbm.at[c, pl.ds(off, _RPS)])

  return deg_kernel(dst, zeros_nw, ones_kw)


def _sc_spmm(hs, src, dst):
  @functools.partial(
      pl.kernel,
      out_type=jax.ShapeDtypeStruct((_NC, _N, _D), jnp.float32),
      mesh=_sc_mesh(),
      scratch_types=[
          pltpu.VMEM((_EPW,), jnp.int32),
          pltpu.VMEM((_NCH, _K), jnp.int32),
          pltpu.VMEM((_K, _D), jnp.float32),
          pltpu.VMEM((_K, _D), jnp.float32),
          pltpu.VMEM_SHARED((_N, _D), jnp.float32),
          pltpu.SemaphoreType.DMA,
          pltpu.SemaphoreType.DMA,
      ],
  )
  def spmm_kernel(hs_hbm, src_hbm, dst_hbm, out_hbm,
                  src_v, dst_v, rows0, rows1, acc_sh, semg0, semg1):
    c = lax.axis_index("c")
    s = lax.axis_index("s")
    wid = s * _NC + c
    off = _row_off(s)
    pltpu.sync_copy(src_hbm.at[wid], src_v)
    pltpu.sync_copy(dst_hbm.at[wid], dst_v)
    pltpu.sync_copy(hs_hbm.at[pl.ds(off, _RPS)],
                    acc_sh.at[pl.ds(off, _RPS)])
    plsc.subcore_barrier()

    def wait_g(j, buf, sem):
      pltpu.make_async_copy(
          hs_hbm.at[src_v.at[pl.ds(j * _K, _K)]], buf, sem).wait()

    def start_g(j, buf, sem):
      pltpu.async_copy(hs_hbm.at[src_v.at[pl.ds(j * _K, _K)]], buf, sem)

    def scat(j, buf):
      pltpu.sync_copy(buf, acc_sh.at[dst_v.at[j]], add=True)

    start_g(0, rows0, semg0)
    wait_g(0, rows0, semg0)
    start_g(1, rows1, semg1)
    scat(0, rows0)

    def pair(p, carry):
      j0 = 2 * p + 1
      j1 = 2 * p + 2
      wait_g(j0, rows1, semg1)
      start_g(j0 + 1, rows0, semg0)
      scat(j0, rows1)
      wait_g(j1, rows0, semg0)
      start_g(jnp.minimum(j1 + 1, _NCH - 1), rows1, semg1)
      scat(j1, rows0)
      return carry

    lax.fori_loop(0, (_NCH - 1) // 2, pair, 0)
    wait_g(_NCH - 1, rows1, semg1)
    plsc.subcore_barrier()
    pltpu.sync_copy(acc_sh.at[pl.ds(off, _RPS)],
                    out_hbm.at[c, pl.ds(off, _RPS)])

  return spmm_kernel(hs, src, dst)


def _tc_pre(x, W1, degs):
  def body(x_ref, w_ref, deg_ref, p_ref, dinv_ref):
    deg = deg_ref[0, :, 0:1] + deg_ref[1, :, 0:1] + 1.0
    dinvb = jnp.broadcast_to(lax.rsqrt(deg), (_N, _D))
    dinv_ref[...] = dinvb
    p_ref[...] = jnp.dot(x_ref[...], w_ref[...],
                         preferred_element_type=jnp.float32) * dinvb

  return pl.pallas_call(
      body,
      out_shape=[jax.ShapeDtypeStruct((_N, _D), jnp.float32),
                 jax.ShapeDtypeStruct((_N, _D), jnp.float32)],
  )(x, W1, degs)


def _tc_mid(accs, P, dinvb, b, g, be, Wn):
  def body(acc_ref, p_ref, dinv_ref, b_ref, g_ref, be_ref, w_ref, out_ref):
    S = acc_ref[0] + acc_ref[1] - p_ref[...]
    T = S * dinv_ref[...] + b_ref[...]
    m = jnp.mean(T, axis=0, keepdims=True)
    v = jnp.mean((T - m) ** 2, axis=0, keepdims=True)
    Hn = g_ref[...] * (T - m) * lax.rsqrt(v + _EPS) + be_ref[...]
    Hn = jnp.maximum(Hn, 0.0)
    out_ref[...] = jnp.dot(Hn, w_ref[...],
                           preferred_element_type=jnp.float32) * dinv_ref[...]

  return pl.pallas_call(
      body,
      out_shape=jax.ShapeDtypeStruct((_N, _D), jnp.float32),
  )(accs, P, dinvb, b, g, be, Wn)


def _tc_fin(accs, P, dinvb, b):
  def body(acc_ref, p_ref, dinv_ref, b_ref, out_ref):
    out_ref[...] = (acc_ref[0] + acc_ref[1] - p_ref[...]) * dinv_ref[...] \
        + b_ref[...]

  return pl.pallas_call(
      body,
      out_shape=jax.ShapeDtypeStruct((_N, _D), jnp.float32),
  )(accs, P, dinvb, b)


def kernel(x, edge_index, W1, b1, W2, b2, W3, b3, g1, be1, g2, be2):
  ei = edge_index.astype(jnp.int32)
  src = ei[0].reshape(_NW, _EPW)
  dst = ei[1].reshape(_NW, _NCH, _K)
  zeros_nw = jnp.zeros((_N, _DEGW), jnp.float32)
  ones_kw = jnp.ones((_K, _DEGW), jnp.float32)

  degs = _sc_degree(dst, zeros_nw, ones_kw)
  P1, dinvb = _tc_pre(x, W1, degs)
  acc1 = _sc_spmm(P1, src, dst)
  P2 = _tc_mid(acc1, P1, dinvb, b1.reshape(1, _D), g1.reshape(1, _D),
               be1.reshape(1, _D), W2)
  acc2 = _sc_spmm(P2, src, dst)
  P3 = _tc_mid(acc2, P2, dinvb, b2.reshape(1, _D), g2.reshape(1, _D),
               be2.reshape(1, _D), W3)
  acc3 = _sc_spmm(P3, src, dst)
  return _tc_fin(acc3, P3, dinvb, b3.reshape(1, _D))


# speedup vs baseline: 21.2326x; 1.0031x over previous
import functools

import jax
import jax.numpy as jnp
from jax import lax
from jax.experimental import pallas as pl
from jax.experimental.pallas import tpu as pltpu
from jax.experimental.pallas import tpu_sc as plsc

_N = 10000
_E = 320000
_D = 128
_EPS = 1e-5
_NC = 2
_NS = 16
_NW = _NC * _NS
_EPW = _E // _NW
_K = 80
_NCH = _EPW // _K
_RPS = 632
_DEGW = 16


def _row_off(s):
  return pl.multiple_of(jnp.minimum(s * _RPS, _N - _RPS), 8)


def _sc_mesh():
  return plsc.VectorSubcoreMesh(
      core_axis_name="c", subcore_axis_name="s",
      num_cores=_NC, num_subcores=_NS)


def _sc_degree(dst, zeros_nw, ones_kw):
  @functools.partial(
      pl.kernel,
      out_type=jax.ShapeDtypeStruct((_NC, _N, _DEGW), jnp.float32),
      mesh=_sc_mesh(),
      compiler_params=pltpu.CompilerParams(use_tc_tiling_on_sc=False),
      scratch_types=[
          pltpu.VMEM((_NCH, _K), jnp.int32),
          pltpu.VMEM((_K, _DEGW), jnp.float32),
          pltpu.VMEM_SHARED((_N, _DEGW), jnp.float32),
      ],
  )
  def deg_kernel(dst_hbm, zero_hbm, one_hbm, out_hbm, dst_v, one_v, acc_sh):
    c = lax.axis_index("c")
    s = lax.axis_index("s")
    wid = s * _NC + c
    off = _row_off(s)
    pltpu.sync_copy(dst_hbm.at[wid], dst_v)
    pltpu.sync_copy(one_hbm, one_v)
    pltpu.sync_copy(zero_hbm.at[pl.ds(off, _RPS)],
                    acc_sh.at[pl.ds(off, _RPS)])
    plsc.subcore_barrier()

    def body(j, carry):
      pltpu.sync_copy(one_v, acc_sh.at[dst_v.at[j]], add=True)
      return carry

    lax.fori_loop(0, _NCH, body, 0)
    plsc.subcore_barrier()
    pltpu.sync_copy(acc_sh.at[pl.ds(off, _RPS)],
                    out_hbm.at[c, pl.ds(off, _RPS)])

  return deg_kernel(dst, zeros_nw, ones_kw)


def _sc_spmm(hs, src, dst):
  @functools.partial(
      pl.kernel,
      out_type=jax.ShapeDtypeStruct((_NC, _N, _D), jnp.float32),
      mesh=_sc_mesh(),
      compiler_params=pltpu.CompilerParams(use_tc_tiling_on_sc=False),
      scratch_types=[
          pltpu.VMEM((_NCH, _K), jnp.int32),
          pltpu.VMEM((_NCH, _K), jnp.int32),
          pltpu.VMEM((_K, _D), jnp.float32),
          pltpu.VMEM((_K, _D), jnp.float32),
          pltpu.VMEM((_K, _D), jnp.float32),
          pltpu.VMEM_SHARED((_N, _D), jnp.float32),
          pltpu.SemaphoreType.DMA,
          pltpu.SemaphoreType.DMA,
          pltpu.SemaphoreType.DMA,
          pltpu.SemaphoreType.DMA,
          pltpu.SemaphoreType.DMA,
          pltpu.SemaphoreType.DMA,
      ],
  )
  def spmm_kernel(hs_hbm, src_hbm, dst_hbm, out_hbm,
                  src_v, dst_v, rows0, rows1, rows2, acc_sh,
                  sg0, sg1, sg2, ss0, ss1, ss2):
    c = lax.axis_index("c")
    s = lax.axis_index("s")
    wid = s * _NC + c
    off = _row_off(s)
    pltpu.sync_copy(src_hbm.at[wid], src_v)
    pltpu.sync_copy(dst_hbm.at[wid], dst_v)
    pltpu.sync_copy(hs_hbm.at[pl.ds(off, _RPS)],
                    acc_sh.at[pl.ds(off, _RPS)])
    plsc.subcore_barrier()

    def wait_g(j, buf, sem):
      pltpu.make_async_copy(hs_hbm.at[src_v.at[j]], buf, sem).wait()

    def start_g(j, buf, sem):
      pltpu.async_copy(hs_hbm.at[src_v.at[j]], buf, sem)

    def start_s(j, buf, sem):
      pltpu.async_copy(buf, acc_sh.at[dst_v.at[j]], sem, add=True)

    def wait_s(buf, sem):
      pltpu.make_async_copy(buf, acc_sh.at[dst_v.at[0]], sem).wait()

    start_g(0, rows0, sg0)
    start_g(1, rows1, sg1)
    start_g(2, rows2, sg2)
    wait_g(0, rows0, sg0)
    start_s(0, rows0, ss0)
    wait_g(1, rows1, sg1)
    start_s(1, rows1, ss1)

    def triple(q, carry):
      t0 = 3 * q + 2
      wait_g(t0, rows2, sg2)
      start_s(t0, rows2, ss2)
      wait_s(rows0, ss0)
      start_g(t0 + 1, rows0, sg0)
      wait_g(t0 + 1, rows0, sg0)
      start_s(t0 + 1, rows0, ss0)
      wait_s(rows1, ss1)
      start_g(t0 + 2, rows1, sg1)
      wait_g(t0 + 2, rows1, sg1)
      start_s(t0 + 2, rows1, ss1)
      wait_s(rows2, ss2)
      start_g(jnp.minimum(t0 + 3, _NCH - 1), rows2, sg2)
      return carry

    lax.fori_loop(0, (_NCH - 2) // 3, triple, 0)
    wait_g(_NCH - 1, rows2, sg2)
    wait_s(rows0, ss0)
    wait_s(rows1, ss1)
    plsc.subcore_barrier()
    pltpu.sync_copy(acc_sh.at[pl.ds(off, _RPS)],
                    out_hbm.at[c, pl.ds(off, _RPS)])

  return spmm_kernel(hs, src, dst)


def _tc_pre(x, W1, degs):
  def body(x_ref, w_ref, deg_ref, p_ref, dinv_ref):
    deg = deg_ref[0, :, 0:1] + deg_ref[1, :, 0:1] + 1.0
    dinvb = jnp.broadcast_to(lax.rsqrt(deg), (_N, _D))
    dinv_ref[...] = dinvb
    p_ref[...] = jnp.dot(x_ref[...], w_ref[...],
                         preferred_element_type=jnp.float32) * dinvb

  return pl.pallas_call(
      body,
      out_shape=[jax.ShapeDtypeStruct((_N, _D), jnp.float32),
                 jax.ShapeDtypeStruct((_N, _D), jnp.float32)],
  )(x, W1, degs)


def _tc_mid(accs, P, dinvb, b, g, be, Wn):
  def body(acc_ref, p_ref, dinv_ref, b_ref, g_ref, be_ref, w_ref, out_ref):
    S = acc_ref[0] + acc_ref[1] - p_ref[...]
    T = S * dinv_ref[...] + b_ref[...]
    m = jnp.mean(T, axis=0, keepdims=True)
    v = jnp.mean((T - m) ** 2, axis=0, keepdims=True)
    Hn = g_ref[...] * (T - m) * lax.rsqrt(v + _EPS) + be_ref[...]
    Hn = jnp.maximum(Hn, 0.0)
    out_ref[...] = jnp.dot(Hn, w_ref[...],
                           preferred_element_type=jnp.float32) * dinv_ref[...]

  return pl.pallas_call(
      body,
      out_shape=jax.ShapeDtypeStruct((_N, _D), jnp.float32),
  )(accs, P, dinvb, b, g, be, Wn)


def _tc_fin(accs, P, dinvb, b):
  def body(acc_ref, p_ref, dinv_ref, b_ref, out_ref):
    out_ref[...] = (acc_ref[0] + acc_ref[1] - p_ref[...]) * dinv_ref[...] \
        + b_ref[...]

  return pl.pallas_call(
      body,
      out_shape=jax.ShapeDtypeStruct((_N, _D), jnp.float32),
  )(accs, P, dinvb, b)


def kernel(x, edge_index, W1, b1, W2, b2, W3, b3, g1, be1, g2, be2):
  ei = edge_index.astype(jnp.int32)
  src = ei[0].reshape(_NW, _NCH, _K)
  dst = ei[1].reshape(_NW, _NCH, _K)
  zeros_nw = jnp.zeros((_N, _DEGW), jnp.float32)
  ones_kw = jnp.ones((_K, _DEGW), jnp.float32)

  degs = _sc_degree(dst, zeros_nw, ones_kw)
  P1, dinvb = _tc_pre(x, W1, degs)
  acc1 = _sc_spmm(P1, src, dst)
  P2 = _tc_mid(acc1, P1, dinvb, b1.reshape(1, _D), g1.reshape(1, _D),
               be1.reshape(1, _D), W2)
  acc2 = _sc_spmm(P2, src, dst)
  P3 = _tc_mid(acc2, P2, dinvb, b2.reshape(1, _D), g2.reshape(1, _D),
               be2.reshape(1, _D), W3)
  acc3 = _sc_spmm(P3, src, dst)
  return _tc_fin(acc3, P3, dinvb, b3.reshape(1, _D))


# speedup vs baseline: 21.4794x; 1.0116x over previous
import functools

import jax
import jax.numpy as jnp
from jax import lax
from jax.experimental import pallas as pl
from jax.experimental.pallas import tpu as pltpu
from jax.experimental.pallas import tpu_sc as plsc

_N = 10000
_E = 320000
_D = 128
_EPS = 1e-5
_NC = 2
_NS = 16
_NW = _NC * _NS
_EPW = _E // _NW
_K = 80
_NCH = _EPW // _K
_RPS = 632
_DEGW = 16


def _row_off(s):
  return pl.multiple_of(jnp.minimum(s * _RPS, _N - _RPS), 8)


def _sc_mesh():
  return plsc.VectorSubcoreMesh(
      core_axis_name="c", subcore_axis_name="s",
      num_cores=_NC, num_subcores=_NS)


def _sc_degree(dst, zeros_nw, ones_kw):
  @functools.partial(
      pl.kernel,
      out_type=jax.ShapeDtypeStruct((_NC, _N, _DEGW), jnp.float32),
      mesh=_sc_mesh(),
      compiler_params=pltpu.CompilerParams(use_tc_tiling_on_sc=False),
      scratch_types=[
          pltpu.VMEM((_NCH, _K), jnp.int32),
          pltpu.VMEM((_K, _DEGW), jnp.float32),
          pltpu.VMEM_SHARED((_N, _DEGW), jnp.float32),
          pltpu.SemaphoreType.DMA,
      ],
  )
  def deg_kernel(dst_hbm, zero_hbm, one_hbm, out_hbm, dst_v, one_v, acc_sh,
                 dsem):
    c = lax.axis_index("c")
    s = lax.axis_index("s")
    wid = s * _NC + c
    off = _row_off(s)
    pltpu.sync_copy(dst_hbm.at[wid], dst_v)
    pltpu.sync_copy(one_hbm, one_v)
    pltpu.sync_copy(zero_hbm.at[pl.ds(off, _RPS)],
                    acc_sh.at[pl.ds(off, _RPS)])
    plsc.subcore_barrier()

    def fire(j, carry):
      pltpu.async_copy(one_v, acc_sh.at[dst_v.at[j]], dsem, add=True)
      return carry

    def drain(j, carry):
      pltpu.make_async_copy(one_v, acc_sh.at[dst_v.at[j]], dsem).wait()
      return carry

    lax.fori_loop(0, _NCH, fire, 0)
    lax.fori_loop(0, _NCH, drain, 0)
    plsc.subcore_barrier()
    pltpu.sync_copy(acc_sh.at[pl.ds(off, _RPS)],
                    out_hbm.at[c, pl.ds(off, _RPS)])

  return deg_kernel(dst, zeros_nw, ones_kw)


def _sc_spmm(hs, zeros_nd, src, dst):
  @functools.partial(
      pl.kernel,
      out_type=jax.ShapeDtypeStruct((_NC, _N, _D), jnp.float32),
      mesh=_sc_mesh(),
      compiler_params=pltpu.CompilerParams(use_tc_tiling_on_sc=False),
      scratch_types=[
          pltpu.VMEM((_NCH, _K), jnp.int32),
          pltpu.VMEM((_NCH, _K), jnp.int32),
          pltpu.VMEM((_K, _D), jnp.float32),
          pltpu.VMEM((_K, _D), jnp.float32),
          pltpu.VMEM((_K, _D), jnp.float32),
          pltpu.VMEM_SHARED((_N, _D), jnp.float32),
          pltpu.SemaphoreType.DMA,
          pltpu.SemaphoreType.DMA,
          pltpu.SemaphoreType.DMA,
          pltpu.SemaphoreType.DMA,
          pltpu.SemaphoreType.DMA,
          pltpu.SemaphoreType.DMA,
      ],
  )
  def spmm_kernel(hs_hbm, zero_hbm, src_hbm, dst_hbm, out_hbm,
                  src_v, dst_v, rows0, rows1, rows2, acc_sh,
                  sg0, sg1, sg2, ss0, ss1, ss2):
    c = lax.axis_index("c")
    s = lax.axis_index("s")
    wid = s * _NC + c
    off = _row_off(s)
    pltpu.sync_copy(src_hbm.at[wid], src_v)
    pltpu.sync_copy(dst_hbm.at[wid], dst_v)

    @pl.when(c == 0)
    def _():
      pltpu.sync_copy(hs_hbm.at[pl.ds(off, _RPS)],
                      acc_sh.at[pl.ds(off, _RPS)])

    @pl.when(c == 1)
    def _():
      pltpu.sync_copy(zero_hbm.at[pl.ds(off, _RPS)],
                      acc_sh.at[pl.ds(off, _RPS)])

    plsc.subcore_barrier()

    def wait_g(j, buf, sem):
      pltpu.make_async_copy(hs_hbm.at[src_v.at[j]], buf, sem).wait()

    def start_g(j, buf, sem):
      pltpu.async_copy(hs_hbm.at[src_v.at[j]], buf, sem)

    def start_s(j, buf, sem):
      pltpu.async_copy(buf, acc_sh.at[dst_v.at[j]], sem, add=True)

    def wait_s(buf, sem):
      pltpu.make_async_copy(buf, acc_sh.at[dst_v.at[0]], sem).wait()

    start_g(0, rows0, sg0)
    start_g(1, rows1, sg1)
    start_g(2, rows2, sg2)
    wait_g(0, rows0, sg0)
    start_s(0, rows0, ss0)
    wait_g(1, rows1, sg1)
    start_s(1, rows1, ss1)

    def triple(q, carry):
      t0 = 3 * q + 2
      wait_g(t0, rows2, sg2)
      start_s(t0, rows2, ss2)
      wait_s(rows0, ss0)
      start_g(t0 + 1, rows0, sg0)
      wait_g(t0 + 1, rows0, sg0)
      start_s(t0 + 1, rows0, ss0)
      wait_s(rows1, ss1)
      start_g(t0 + 2, rows1, sg1)
      wait_g(t0 + 2, rows1, sg1)
      start_s(t0 + 2, rows1, ss1)
      wait_s(rows2, ss2)
      start_g(jnp.minimum(t0 + 3, _NCH - 1), rows2, sg2)
      return carry

    lax.fori_loop(0, (_NCH - 2) // 3, triple, 0)
    wait_g(_NCH - 1, rows2, sg2)
    wait_s(rows0, ss0)
    wait_s(rows1, ss1)
    plsc.subcore_barrier()
    pltpu.sync_copy(acc_sh.at[pl.ds(off, _RPS)],
                    out_hbm.at[c, pl.ds(off, _RPS)])

  return spmm_kernel(hs, zeros_nd, src, dst)


def _tc_pre(x, W1, degs):
  def body(x_ref, w_ref, deg_ref, p_ref, dinv_ref):
    deg = deg_ref[0, :, 0:1] + deg_ref[1, :, 0:1] + 1.0
    dinv = lax.rsqrt(deg)
    dinv_ref[...] = jnp.broadcast_to(dinv, (_N, 8))
    p_ref[...] = jnp.dot(x_ref[...], w_ref[...],
                         preferred_element_type=jnp.float32) \
        * jnp.broadcast_to(dinv, (_N, _D))

  return pl.pallas_call(
      body,
      out_shape=[jax.ShapeDtypeStruct((_N, _D), jnp.float32),
                 jax.ShapeDtypeStruct((_N, 8), jnp.float32)],
  )(x, W1, degs)


def _tc_mid(accs, dinv8, b, g, be, Wn):
  def body(acc_ref, dinv_ref, b_ref, g_ref, be_ref, w_ref, out_ref):
    dinvb = jnp.broadcast_to(dinv_ref[:, 0:1], (_N, _D))
    T = (acc_ref[0] + acc_ref[1]) * dinvb + b_ref[...]
    m = jnp.mean(T, axis=0, keepdims=True)
    v = jnp.mean((T - m) ** 2, axis=0, keepdims=True)
    Hn = g_ref[...] * (T - m) * lax.rsqrt(v + _EPS) + be_ref[...]
    Hn = jnp.maximum(Hn, 0.0)
    out_ref[...] = jnp.dot(Hn, w_ref[...],
                           preferred_element_type=jnp.float32) * dinvb

  return pl.pallas_call(
      body,
      out_shape=jax.ShapeDtypeStruct((_N, _D), jnp.float32),
  )(accs, dinv8, b, g, be, Wn)


def _tc_fin(accs, dinv8, b):
  def body(acc_ref, dinv_ref, b_ref, out_ref):
    dinvb = jnp.broadcast_to(dinv_ref[:, 0:1], (_N, _D))
    out_ref[...] = (acc_ref[0] + acc_ref[1]) * dinvb + b_ref[...]

  return pl.pallas_call(
      body,
      out_shape=jax.ShapeDtypeStruct((_N, _D), jnp.float32),
  )(accs, dinv8, b)


def kernel(x, edge_index, W1, b1, W2, b2, W3, b3, g1, be1, g2, be2):
  ei = edge_index.astype(jnp.int32)
  src = ei[0].reshape(_NW, _NCH, _K)
  dst = ei[1].reshape(_NW, _NCH, _K)
  zeros_nw = jnp.zeros((_N, _DEGW), jnp.float32)
  ones_kw = jnp.ones((_K, _DEGW), jnp.float32)
  zeros_nd = jnp.zeros((_N, _D), jnp.float32)

  degs = _sc_degree(dst, zeros_nw, ones_kw)
  P1, dinv8 = _tc_pre(x, W1, degs)
  acc1 = _sc_spmm(P1, zeros_nd, src, dst)
  P2 = _tc_mid(acc1, dinv8, b1.reshape(1, _D), g1.reshape(1, _D),
               be1.reshape(1, _D), W2)
  acc2 = _sc_spmm(P2, zeros_nd, src, dst)
  P3 = _tc_mid(acc2, dinv8, b2.reshape(1, _D), g2.reshape(1, _D),
               be2.reshape(1, _D), W3)
  acc3 = _sc_spmm(P3, zeros_nd, src, dst)
  return _tc_fin(acc3, dinv8, b3.reshape(1, _D))


# speedup vs baseline: 21.7943x; 1.0147x over previous
import functools

import jax
import jax.numpy as jnp
from jax import lax
from jax.experimental import pallas as pl
from jax.experimental.pallas import tpu as pltpu
from jax.experimental.pallas import tpu_sc as plsc

_N = 10000
_E = 320000
_D = 128
_EPS = 1e-5
_NC = 2
_NS = 16
_NW = _NC * _NS
_EPW = _E // _NW
_K = 80
_NCH = _EPW // _K
_RPS = 632
_DEGW = 16


def _row_off(s):
  return pl.multiple_of(jnp.minimum(s * _RPS, _N - _RPS), 8)


def _sc_mesh():
  return plsc.VectorSubcoreMesh(
      core_axis_name="c", subcore_axis_name="s",
      num_cores=_NC, num_subcores=_NS)


def _sc_degree(dst, zeros_nw, ones_kw):
  @functools.partial(
      pl.kernel,
      out_type=jax.ShapeDtypeStruct((_NC, _N, _DEGW), jnp.float32),
      mesh=_sc_mesh(),
      compiler_params=pltpu.CompilerParams(use_tc_tiling_on_sc=False),
      scratch_types=[
          pltpu.VMEM((_NCH, _K), jnp.int32),
          pltpu.VMEM((_K, _DEGW), jnp.float32),
          pltpu.VMEM_SHARED((_N, _DEGW), jnp.float32),
          pltpu.SemaphoreType.DMA,
          pltpu.SemaphoreType.DMA,
          pltpu.SemaphoreType.DMA,
      ],
  )
  def deg_kernel(dst_hbm, zero_hbm, one_hbm, out_hbm, dst_v, one_v, acc_sh,
                 dsem, dsem2, dsem3):
    c = lax.axis_index("c")
    s = lax.axis_index("s")
    wid = s * _NC + c
    off = _row_off(s)
    cp1 = pltpu.async_copy(dst_hbm.at[wid], dst_v, dsem)
    cp2 = pltpu.async_copy(one_hbm, one_v, dsem2)
    cp3 = pltpu.async_copy(zero_hbm.at[pl.ds(off, _RPS)],
                           acc_sh.at[pl.ds(off, _RPS)], dsem3)
    cp1.wait()
    cp2.wait()
    cp3.wait()
    plsc.subcore_barrier()

    def fire(j, carry):
      pltpu.async_copy(one_v, acc_sh.at[dst_v.at[j]], dsem, add=True)
      return carry

    def drain(j, carry):
      pltpu.make_async_copy(one_v, acc_sh.at[dst_v.at[j]], dsem).wait()
      return carry

    lax.fori_loop(0, _NCH, fire, 0)
    lax.fori_loop(0, _NCH, drain, 0)
    plsc.subcore_barrier()
    pltpu.sync_copy(acc_sh.at[pl.ds(off, _RPS)],
                    out_hbm.at[c, pl.ds(off, _RPS)])

  return deg_kernel(dst, zeros_nw, ones_kw)


def _sc_spmm(hs, zeros_nd, src, dst):
  @functools.partial(
      pl.kernel,
      out_type=jax.ShapeDtypeStruct((_NC, _N, _D), jnp.float32),
      mesh=_sc_mesh(),
      compiler_params=pltpu.CompilerParams(use_tc_tiling_on_sc=False),
      scratch_types=[
          pltpu.VMEM((_NCH, _K), jnp.int32),
          pltpu.VMEM((_NCH, _K), jnp.int32),
          pltpu.VMEM((_K, _D), jnp.float32),
          pltpu.VMEM((_K, _D), jnp.float32),
          pltpu.VMEM((_K, _D), jnp.float32),
          pltpu.VMEM_SHARED((_N, _D), jnp.float32),
          pltpu.SemaphoreType.DMA,
          pltpu.SemaphoreType.DMA,
          pltpu.SemaphoreType.DMA,
          pltpu.SemaphoreType.DMA,
          pltpu.SemaphoreType.DMA,
          pltpu.SemaphoreType.DMA,
      ],
  )
  def spmm_kernel(hs_hbm, zero_hbm, src_hbm, dst_hbm, out_hbm,
                  src_v, dst_v, rows0, rows1, rows2, acc_sh,
                  sg0, sg1, sg2, ss0, ss1, ss2):
    c = lax.axis_index("c")
    s = lax.axis_index("s")
    wid = s * _NC + c
    off = _row_off(s)
    cp_src = pltpu.async_copy(src_hbm.at[wid], src_v, sg0)
    cp_dst = pltpu.async_copy(dst_hbm.at[wid], dst_v, sg1)

    @pl.when(c == 0)
    def _():
      pltpu.async_copy(hs_hbm.at[pl.ds(off, _RPS)],
                       acc_sh.at[pl.ds(off, _RPS)], sg2)

    @pl.when(c == 1)
    def _():
      pltpu.async_copy(zero_hbm.at[pl.ds(off, _RPS)],
                       acc_sh.at[pl.ds(off, _RPS)], sg2)

    cp_src.wait()
    cp_dst.wait()
    pltpu.make_async_copy(hs_hbm.at[pl.ds(off, _RPS)],
                          acc_sh.at[pl.ds(off, _RPS)], sg2).wait()
    plsc.subcore_barrier()

    def wait_g(j, buf, sem):
      pltpu.make_async_copy(hs_hbm.at[src_v.at[j]], buf, sem).wait()

    def start_g(j, buf, sem):
      pltpu.async_copy(hs_hbm.at[src_v.at[j]], buf, sem)

    def start_s(j, buf, sem):
      pltpu.async_copy(buf, acc_sh.at[dst_v.at[j]], sem, add=True)

    def wait_s(buf, sem):
      pltpu.make_async_copy(buf, acc_sh.at[dst_v.at[0]], sem).wait()

    start_g(0, rows0, sg0)
    start_g(1, rows1, sg1)
    start_g(2, rows2, sg2)
    wait_g(0, rows0, sg0)
    start_s(0, rows0, ss0)
    wait_g(1, rows1, sg1)
    start_s(1, rows1, ss1)

    def triple(q, carry):
      t0 = 3 * q + 2
      wait_g(t0, rows2, sg2)
      start_s(t0, rows2, ss2)
      wait_s(rows0, ss0)
      start_g(t0 + 1, rows0, sg0)
      wait_g(t0 + 1, rows0, sg0)
      start_s(t0 + 1, rows0, ss0)
      wait_s(rows1, ss1)
      start_g(t0 + 2, rows1, sg1)
      wait_g(t0 + 2, rows1, sg1)
      start_s(t0 + 2, rows1, ss1)
      wait_s(rows2, ss2)
      start_g(jnp.minimum(t0 + 3, _NCH - 1), rows2, sg2)
      return carry

    lax.fori_loop(0, (_NCH - 2) // 3, triple, 0)
    wait_g(_NCH - 1, rows2, sg2)
    wait_s(rows0, ss0)
    wait_s(rows1, ss1)
    plsc.subcore_barrier()
    pltpu.sync_copy(acc_sh.at[pl.ds(off, _RPS)],
                    out_hbm.at[c, pl.ds(off, _RPS)])

  return spmm_kernel(hs, zeros_nd, src, dst)


def _tc_pre(x, W1, degs):
  def body(x_ref, w_ref, deg_ref, p_ref, dinv_ref):
    deg = deg_ref[0, :, 0:1] + deg_ref[1, :, 0:1] + 1.0
    dinv = lax.rsqrt(deg)
    dinv_ref[...] = jnp.broadcast_to(dinv, (_N, 8))
    p_ref[...] = jnp.dot(x_ref[...], w_ref[...],
                         preferred_element_type=jnp.float32) \
        * jnp.broadcast_to(dinv, (_N, _D))

  return pl.pallas_call(
      body,
      out_shape=[jax.ShapeDtypeStruct((_N, _D), jnp.float32),
                 jax.ShapeDtypeStruct((_N, 8), jnp.float32)],
  )(x, W1, degs)


def _tc_mid(accs, dinv8, b, g, be, Wn):
  def body(acc_ref, dinv_ref, b_ref, g_ref, be_ref, w_ref, out_ref):
    dinvb = jnp.broadcast_to(dinv_ref[:, 0:1], (_N, _D))
    T = (acc_ref[0] + acc_ref[1]) * dinvb + b_ref[...]
    m = jnp.mean(T, axis=0, keepdims=True)
    v = jnp.mean((T - m) ** 2, axis=0, keepdims=True)
    Hn = g_ref[...] * (T - m) * lax.rsqrt(v + _EPS) + be_ref[...]
    Hn = jnp.maximum(Hn, 0.0)
    out_ref[...] = jnp.dot(Hn, w_ref[...],
                           preferred_element_type=jnp.float32) * dinvb

  return pl.pallas_call(
      body,
      out_shape=jax.ShapeDtypeStruct((_N, _D), jnp.float32),
  )(accs, dinv8, b, g, be, Wn)


def _tc_fin(accs, dinv8, b):
  def body(acc_ref, dinv_ref, b_ref, out_ref):
    dinvb = jnp.broadcast_to(dinv_ref[:, 0:1], (_N, _D))
    out_ref[...] = (acc_ref[0] + acc_ref[1]) * dinvb + b_ref[...]

  return pl.pallas_call(
      body,
      out_shape=jax.ShapeDtypeStruct((_N, _D), jnp.float32),
  )(accs, dinv8, b)


def kernel(x, edge_index, W1, b1, W2, b2, W3, b3, g1, be1, g2, be2):
  ei = edge_index.astype(jnp.int32)
  src = ei[0].reshape(_NW, _NCH, _K)
  dst = ei[1].reshape(_NW, _NCH, _K)
  zeros_nw = jnp.zeros((_N, _DEGW), jnp.float32)
  ones_kw = jnp.ones((_K, _DEGW), jnp.float32)
  zeros_nd = jnp.zeros((_N, _D), jnp.float32)

  degs = _sc_degree(dst, zeros_nw, ones_kw)
  P1, dinv8 = _tc_pre(x, W1, degs)
  acc1 = _sc_spmm(P1, zeros_nd, src, dst)
  P2 = _tc_mid(acc1, dinv8, b1.reshape(1, _D), g1.reshape(1, _D),
               be1.reshape(1, _D), W2)
  acc2 = _sc_spmm(P2, zeros_nd, src, dst)
  P3 = _tc_mid(acc2, dinv8, b2.reshape(1, _D), g2.reshape(1, _D),
               be2.reshape(1, _D), W3)
  acc3 = _sc_spmm(P3, zeros_nd, src, dst)
  return _tc_fin(acc3, dinv8, b3.reshape(1, _D))
